# baseline (speedup 1.0000x reference)
import functools

import jax
import jax.numpy as jnp
from jax import lax
from jax.experimental import pallas as pl
from jax.experimental.pallas import tpu as pltpu
from jax.experimental.pallas import tpu_sc as plsc

_D = 64
_NW = 32
_BB = 128
_L = 16


@functools.lru_cache(maxsize=None)
def _make_gather(batch: int, seq: int, vrows: int):
    nblk = batch // _BB
    blk_per_w = nblk // _NW
    chunks_per_w = blk_per_w * seq
    mesh = plsc.VectorSubcoreMesh(core_axis_name="c", subcore_axis_name="s")

    scratch = [
        pltpu.VMEM((_BB // 2, seq), jnp.int32),
        pltpu.VMEM((chunks_per_w, _BB), jnp.int32),
        pltpu.VMEM((chunks_per_w, _BB), jnp.int32),
        pltpu.VMEM((_BB, 2 * _D), jnp.float32),
        pltpu.VMEM((_BB, 2 * _D), jnp.float32),
        pltpu.VMEM((1, _D, _BB), jnp.float32),
        pltpu.VMEM((1, _D, _BB), jnp.float32),
        pltpu.SemaphoreType.DMA,
        pltpu.SemaphoreType.DMA,
        pltpu.SemaphoreType.DMA,
        pltpu.SemaphoreType.DMA,
    ]

    @functools.partial(
        pl.kernel,
        mesh=mesh,
        compiler_params=pltpu.CompilerParams(use_tc_tiling_on_sc=True,
                                             needs_layout_passes=False),
        out_type=jax.ShapeDtypeStruct((seq, _D, batch), jnp.float32),
        scratch_types=scratch,
    )
    def gather_kernel(sent_hbm, table2_hbm, out_hbm, idx_raw, idx2, colb,
                      gb0, gb1, ob0, ob1, gs0, gs1, ws0, ws1):
        wid = lax.axis_index("s") * 2 + lax.axis_index("c")
        blk0 = wid * blk_per_w
        rows = [lax.iota(jnp.int32, _L) + _L * g for g in range(_BB // _L)]

        for hb in range(2 * blk_per_w):
            b0 = blk0 * _BB + hb * (_BB // 2)
            pltpu.sync_copy(sent_hbm.at[pl.ds(b0, _BB // 2)], idx_raw)

            def prep_row(s, carry, hb=hb):
                col = jnp.full((_L,), 0, jnp.int32) + s
                crow = (hb // 2) * seq + s
                coff = (hb % 2) * (_BB // 2)
                for g in range(_BB // (2 * _L)):
                    ids = plsc.load_gather(idx_raw, [rows[g], col])
                    idx2[crow, pl.ds(coff + g * _L, _L)] = ids >> 1
                    colb[crow, pl.ds(coff + g * _L, _L)] = (ids & 1) << 6
                return carry

            lax.fori_loop(0, seq, prep_row, 0)

        gbufs, obufs = (gb0, gb1), (ob0, ob1)
        gsems, wsems = (gs0, gs1), (ws0, ws1)

        def out_box(k):
            bb = k // seq
            s = k - bb * seq
            b0 = (blk0 + bb) * _BB
            return out_hbm.at[pl.ds(s, 1), :, pl.ds(b0, _BB)]

        pltpu.async_copy(table2_hbm.at[idx2.at[0]], gb0, gs0)
        pltpu.async_copy(table2_hbm.at[idx2.at[1]], gb1, gs1)

        def round_body(p, carry):
            for j in range(2):
                k = 2 * p + j
                gb, ob = gbufs[j], obufs[j]
                pltpu.make_async_copy(table2_hbm.at[idx2.at[k]], gb,
                                      gsems[j]).wait()

                @pl.when(p > 0)
                def _(j=j, k=k, ob=ob):
                    pltpu.make_async_copy(ob, out_box(k - 2), wsems[j]).wait()

                cbases = [colb[k, pl.ds(g * _L, _L)]
                          for g in range(_BB // _L)]
                zero = jnp.zeros((_L,), jnp.int32)

                def d_block(i, carry, gb=gb, ob=ob, cbases=cbases):
                    for dd in range(8):
                        dvec = (lax.iota(jnp.int32, _L) + (i * 8 + dd)) \
                            & (_D - 1)
                        vals = [plsc.load_gather(gb,
                                                 [rows[g], cbases[g] + dvec])
                                for g in range(_BB // _L)]
                        for g in range(_BB // _L):
                            plsc.store_scatter(ob, [zero, dvec, rows[g]],
                                               vals[g])
                    return carry

                lax.fori_loop(0, _D // 8, d_block, 0)
                pltpu.async_copy(ob, out_box(k), wsems[j])

                @pl.when(p < chunks_per_w // 2 - 1)
                def _(j=j, k=k, gb=gb):
                    pltpu.async_copy(table2_hbm.at[idx2.at[k + 2]], gb,
                                     gsems[j])
            return carry

        lax.fori_loop(0, chunks_per_w // 2, round_body, 0)
        for j in range(2):
            pltpu.make_async_copy(obufs[j], out_box(chunks_per_w - 2 + j),
                                  wsems[j]).wait()

    return gather_kernel


def kernel(sentence, word_vectors):
    batch, seq = sentence.shape
    vocab = word_vectors.shape[0]
    vpad = -vocab % 16
    table2 = jnp.pad(word_vectors, ((0, vpad), (0, 0))).reshape(-1, 2 * _D)
    idx = sentence.astype(jnp.int32)
    out = _make_gather(batch, seq, table2.shape[0])(idx, table2)
    return jnp.transpose(out, (2, 0, 1))
